# baseline (speedup 1.0000x reference)
import functools

import jax
import jax.numpy as jnp
from jax import lax
from jax.experimental import pallas as pl
from jax.experimental.pallas import tpu as pltpu
from jax.experimental.pallas import tpu_sc as plsc


def _expand_body(embt_ref, out_ref, *, D, C):
    x = embt_ref[...]
    xt = jnp.transpose(x, (1, 0)).astype(jnp.bfloat16)
    out_ref[...] = jnp.concatenate(
        [xt, jnp.zeros((C, 128 - D), jnp.bfloat16)], axis=1
    )


def _expand_call(embt, *, C=4096):
    D, V = embt.shape
    return pl.pallas_call(
        functools.partial(_expand_body, D=D, C=C),
        grid=(pl.cdiv(V, C),),
        in_specs=[pl.BlockSpec((D, C), lambda i: (0, i))],
        out_specs=pl.BlockSpec((C, 128), lambda i: (i, 0)),
        out_shape=jax.ShapeDtypeStruct((V, 128), jnp.bfloat16),
        compiler_params=pltpu.CompilerParams(
            dimension_semantics=("arbitrary",),
        ),
    )(embt)


@functools.lru_cache(maxsize=None)
def _make_sc_gather(V, B, T):
    info = plsc.get_sparse_core_info()
    NC, NS = info.num_cores, info.num_subcores
    NW = NC * NS
    BB = B // NW
    assert BB == 128
    mesh = plsc.VectorSubcoreMesh(core_axis_name="c", subcore_axis_name="s")

    @functools.partial(
        pl.kernel,
        mesh=mesh,
        out_type=jax.ShapeDtypeStruct((T, B, 128), jnp.bfloat16),
        scratch_types=[
            pltpu.VMEM((T, BB), jnp.int32),
            pltpu.VMEM((2, BB, 128), jnp.bfloat16),
            pltpu.SemaphoreType.DMA,
            pltpu.SemaphoreType.DMA,
        ],
        compiler_params=pltpu.CompilerParams(use_tc_tiling_on_sc=False),
    )
    def gather_k(embp_hbm, idxt_hbm, out_hbm, idx_v, rows_v, sem0, sem1):
        wid = lax.axis_index("s") * NC + lax.axis_index("c")
        b0 = wid * BB
        pltpu.sync_copy(idxt_hbm.at[:, pl.ds(b0, BB)], idx_v)

        def fire(t, buf, sem):
            pltpu.async_copy(embp_hbm.at[idx_v.at[t]], rows_v.at[buf], sem)

        def drain(t, buf, sem):
            pltpu.make_async_copy(
                embp_hbm.at[idx_v.at[t]], rows_v.at[buf], sem
            ).wait()
            pltpu.sync_copy(rows_v.at[buf], out_hbm.at[t, pl.ds(b0, BB)])

        fire(0, 0, sem0)

        def body(i, carry):
            t = 2 * i
            fire(t + 1, 1, sem1)
            drain(t, 0, sem0)
            fire(t + 2, 0, sem0)
            drain(t + 1, 1, sem1)
            return carry

        lax.fori_loop(0, T // 2 - 1, body, 0)
        t = T - 2
        fire(t + 1, 1, sem1)
        drain(t, 0, sem0)
        drain(t + 1, 1, sem1)

    return gather_k


def _lstm_body(x_ref, wih_ref, whh_ref, b_ref, wfc_ref, bfc_ref, out_ref,
               h_ref, c_ref, *, H, T, D):
    t = pl.program_id(1)

    @pl.when(t == 0)
    def _():
        h_ref[...] = jnp.zeros_like(h_ref)
        c_ref[...] = jnp.zeros_like(c_ref)

    x = x_ref[0][:, 0:D].astype(jnp.float32)
    h = h_ref[...]
    g = (
        jnp.dot(x, wih_ref[...], preferred_element_type=jnp.float32)
        + jnp.dot(h, whh_ref[...], preferred_element_type=jnp.float32)
        + b_ref[...]
    )
    i_g = jax.nn.sigmoid(g[:, 0 * H:1 * H])
    f_g = jax.nn.sigmoid(g[:, 1 * H:2 * H])
    g_g = jnp.tanh(g[:, 2 * H:3 * H])
    o_g = jax.nn.sigmoid(g[:, 3 * H:4 * H])
    c_new = f_g * c_ref[...] + i_g * g_g
    h_new = o_g * jnp.tanh(c_new)
    c_ref[...] = c_new
    h_ref[...] = h_new

    @pl.when(t == T - 1)
    def _():
        logits = (
            jnp.dot(h_new, wfc_ref[...], preferred_element_type=jnp.float32)
            + bfc_ref[...]
        )
        m = jnp.max(logits, axis=-1, keepdims=True)
        s = logits - m
        lse = jnp.log(jnp.sum(jnp.exp(s), axis=-1, keepdims=True))
        out_ref[...] = s - lse


def _lstm_call(x, wih, whh, b1, wfc, bfc1, *, bt=512):
    T, B, _ = x.shape
    D = wih.shape[0]
    H = whh.shape[0]
    A = wfc.shape[1]
    grid = (B // bt, T)
    return pl.pallas_call(
        functools.partial(_lstm_body, H=H, T=T, D=D),
        grid=grid,
        in_specs=[
            pl.BlockSpec((1, bt, 128), lambda b, t: (t, b, 0)),
            pl.BlockSpec((D, 4 * H), lambda b, t: (0, 0)),
            pl.BlockSpec((H, 4 * H), lambda b, t: (0, 0)),
            pl.BlockSpec((1, 4 * H), lambda b, t: (0, 0)),
            pl.BlockSpec((H, A), lambda b, t: (0, 0)),
            pl.BlockSpec((1, A), lambda b, t: (0, 0)),
        ],
        out_specs=pl.BlockSpec((bt, A), lambda b, t: (b, 0)),
        out_shape=jax.ShapeDtypeStruct((B, A), jnp.float32),
        scratch_shapes=[
            pltpu.VMEM((bt, H), jnp.float32),
            pltpu.VMEM((bt, H), jnp.float32),
        ],
        compiler_params=pltpu.CompilerParams(
            dimension_semantics=("parallel", "arbitrary"),
        ),
    )(x, wih, whh, b1, wfc, bfc1)


def kernel(dct_in, emb, W_ih, W_hh, b_ih, b_hh, W_fc, b_fc):
    B, T = dct_in.shape
    V, D = emb.shape
    H = W_hh.shape[1]
    A = W_fc.shape[0]

    embp = _expand_call(emb.T)
    idxt = dct_in.T.astype(jnp.int32)
    x3 = _make_sc_gather(V, B, T)(embp, idxt)

    b1 = (b_ih + b_hh).reshape(1, 4 * H)
    bfc1 = b_fc.reshape(1, A)
    return _lstm_call(x3, W_ih.T, W_hh.T, b1, W_fc.T, bfc1)


# speedup vs baseline: 2.9412x; 2.9412x over previous
import functools

import jax
import jax.numpy as jnp
from jax import lax
from jax.experimental import pallas as pl
from jax.experimental.pallas import tpu as pltpu
from jax.experimental.pallas import tpu_sc as plsc


def _expand_body(embt_ref, out_ref, *, D, C):
    x = embt_ref[...]
    xt = jnp.transpose(x, (1, 0))
    out_ref[...] = jnp.concatenate(
        [xt, jnp.zeros((C, 128 - D), jnp.float32)], axis=1
    )


def _expand_call(embt, *, C=8192):
    D, V = embt.shape
    return pl.pallas_call(
        functools.partial(_expand_body, D=D, C=C),
        grid=(pl.cdiv(V, C),),
        in_specs=[pl.BlockSpec((D, C), lambda i: (0, i))],
        out_specs=pl.BlockSpec((C, 128), lambda i: (i, 0)),
        out_shape=jax.ShapeDtypeStruct((V, 128), jnp.float32),
        compiler_params=pltpu.CompilerParams(
            dimension_semantics=("arbitrary",),
        ),
    )(embt)


@functools.lru_cache(maxsize=None)
def _make_sc_gather(V, B, T):
    info = plsc.get_sparse_core_info()
    NC, NS = info.num_cores, info.num_subcores
    NW = NC * NS
    BB = B // NW
    assert BB == 128
    mesh = plsc.VectorSubcoreMesh(core_axis_name="c", subcore_axis_name="s")

    @functools.partial(
        pl.kernel,
        mesh=mesh,
        out_type=jax.ShapeDtypeStruct((T, B, 128), jnp.float32),
        scratch_types=[
            pltpu.VMEM((T, BB), jnp.int32),
            pltpu.VMEM((2, BB, 128), jnp.float32),
            pltpu.SemaphoreType.DMA,
            pltpu.SemaphoreType.DMA,
        ],
        compiler_params=pltpu.CompilerParams(use_tc_tiling_on_sc=False),
    )
    def gather_k(embp_hbm, idxt_hbm, out_hbm, idx_v, rows_v, sem0, sem1):
        wid = lax.axis_index("s") * NC + lax.axis_index("c")
        b0 = wid * BB
        pltpu.sync_copy(idxt_hbm.at[:, pl.ds(b0, BB)], idx_v)

        def fire(t, buf, sem):
            pltpu.async_copy(embp_hbm.at[idx_v.at[t]], rows_v.at[buf], sem)

        def drain(t, buf, sem):
            pltpu.make_async_copy(
                embp_hbm.at[idx_v.at[t]], rows_v.at[buf], sem
            ).wait()
            pltpu.sync_copy(rows_v.at[buf], out_hbm.at[t, pl.ds(b0, BB)])

        fire(0, 0, sem0)

        def body(i, carry):
            t = 2 * i
            fire(t + 1, 1, sem1)
            drain(t, 0, sem0)
            fire(t + 2, 0, sem0)
            drain(t + 1, 1, sem1)
            return carry

        lax.fori_loop(0, T // 2 - 1, body, 0)
        t = T - 2
        fire(t + 1, 1, sem1)
        drain(t, 0, sem0)
        drain(t + 1, 1, sem1)

    return gather_k


def _lstm_body(x_ref, wih_ref, whh_ref, b_ref, wfc_ref, bfc_ref, out_ref,
               h_ref, c_ref, *, H, T, D):
    t = pl.program_id(1)

    @pl.when(t == 0)
    def _():
        h_ref[...] = jnp.zeros_like(h_ref)
        c_ref[...] = jnp.zeros_like(c_ref)

    x = x_ref[0][:, 0:D]
    h = h_ref[...]
    g = (
        jnp.dot(x, wih_ref[...], preferred_element_type=jnp.float32)
        + jnp.dot(h, whh_ref[...], preferred_element_type=jnp.float32)
        + b_ref[...]
    )
    i_g = jax.nn.sigmoid(g[:, 0 * H:1 * H])
    f_g = jax.nn.sigmoid(g[:, 1 * H:2 * H])
    g_g = jnp.tanh(g[:, 2 * H:3 * H])
    o_g = jax.nn.sigmoid(g[:, 3 * H:4 * H])
    c_new = f_g * c_ref[...] + i_g * g_g
    h_new = o_g * jnp.tanh(c_new)
    c_ref[...] = c_new
    h_ref[...] = h_new

    @pl.when(t == T - 1)
    def _():
        logits = (
            jnp.dot(h_new, wfc_ref[...], preferred_element_type=jnp.float32)
            + bfc_ref[...]
        )
        m = jnp.max(logits, axis=-1, keepdims=True)
        s = logits - m
        lse = jnp.log(jnp.sum(jnp.exp(s), axis=-1, keepdims=True))
        out_ref[...] = s - lse


def _lstm_call(x, wih, whh, b1, wfc, bfc1, *, bt=1024):
    T, B, _ = x.shape
    D = wih.shape[0]
    H = whh.shape[0]
    A = wfc.shape[1]
    grid = (B // bt, T)
    return pl.pallas_call(
        functools.partial(_lstm_body, H=H, T=T, D=D),
        grid=grid,
        in_specs=[
            pl.BlockSpec((1, bt, 128), lambda b, t: (t, b, 0)),
            pl.BlockSpec((D, 4 * H), lambda b, t: (0, 0)),
            pl.BlockSpec((H, 4 * H), lambda b, t: (0, 0)),
            pl.BlockSpec((1, 4 * H), lambda b, t: (0, 0)),
            pl.BlockSpec((H, A), lambda b, t: (0, 0)),
            pl.BlockSpec((1, A), lambda b, t: (0, 0)),
        ],
        out_specs=pl.BlockSpec((bt, A), lambda b, t: (b, 0)),
        out_shape=jax.ShapeDtypeStruct((B, A), jnp.float32),
        scratch_shapes=[
            pltpu.VMEM((bt, H), jnp.float32),
            pltpu.VMEM((bt, H), jnp.float32),
        ],
        compiler_params=pltpu.CompilerParams(
            dimension_semantics=("parallel", "arbitrary"),
        ),
    )(x, wih, whh, b1, wfc, bfc1)


def kernel(dct_in, emb, W_ih, W_hh, b_ih, b_hh, W_fc, b_fc):
    B, T = dct_in.shape
    V, D = emb.shape
    H = W_hh.shape[1]
    A = W_fc.shape[0]

    embp = _expand_call(emb.T)
    idxt = dct_in.T.astype(jnp.int32)
    x3 = _make_sc_gather(V, B, T)(embp, idxt)

    b1 = (b_ih + b_hh).reshape(1, 4 * H)
    bfc1 = b_fc.reshape(1, A)
    return _lstm_call(x3, W_ih.T, W_hh.T, b1, W_fc.T, bfc1)


# speedup vs baseline: 3.2042x; 1.0894x over previous
import functools

import jax
import jax.numpy as jnp
from jax import lax
from jax.experimental import pallas as pl
from jax.experimental.pallas import tpu as pltpu
from jax.experimental.pallas import tpu_sc as plsc


def _expand_body(embt_ref, out_ref, *, D, C):
    x = embt_ref[...]
    xt = jnp.transpose(x, (1, 0))
    out_ref[...] = jnp.concatenate(
        [xt, jnp.zeros((C, 128 - D), jnp.float32)], axis=1
    )


def _expand_call(embt, *, C=16384):
    D, V = embt.shape
    return pl.pallas_call(
        functools.partial(_expand_body, D=D, C=C),
        grid=(pl.cdiv(V, C),),
        in_specs=[pl.BlockSpec((D, C), lambda i: (0, i))],
        out_specs=pl.BlockSpec((C, 128), lambda i: (i, 0)),
        out_shape=jax.ShapeDtypeStruct((V, 128), jnp.float32),
        compiler_params=pltpu.CompilerParams(
            dimension_semantics=("arbitrary",),
        ),
    )(embt)


@functools.lru_cache(maxsize=None)
def _make_sc_gather(V, B, T):
    info = plsc.get_sparse_core_info()
    NC, NS = info.num_cores, info.num_subcores
    NW = NC * NS
    BB = B // NW
    assert BB == 128
    mesh = plsc.VectorSubcoreMesh(core_axis_name="c", subcore_axis_name="s")

    @functools.partial(
        pl.kernel,
        mesh=mesh,
        out_type=jax.ShapeDtypeStruct((T, B, 128), jnp.float32),
        scratch_types=[
            pltpu.VMEM((T, BB), jnp.int32),
            pltpu.VMEM((2, BB, 128), jnp.float32),
            pltpu.SemaphoreType.DMA,
            pltpu.SemaphoreType.DMA,
        ],
        compiler_params=pltpu.CompilerParams(use_tc_tiling_on_sc=False),
    )
    def gather_k(embp_hbm, idxt_hbm, out_hbm, idx_v, rows_v, sem0, sem1):
        wid = lax.axis_index("s") * NC + lax.axis_index("c")
        b0 = wid * BB
        pltpu.sync_copy(idxt_hbm.at[:, pl.ds(b0, BB)], idx_v)

        def fire(t, buf, sem):
            pltpu.async_copy(embp_hbm.at[idx_v.at[t]], rows_v.at[buf], sem)

        def drain(t, buf, sem):
            pltpu.make_async_copy(
                embp_hbm.at[idx_v.at[t]], rows_v.at[buf], sem
            ).wait()
            pltpu.sync_copy(rows_v.at[buf], out_hbm.at[t, pl.ds(b0, BB)])

        fire(0, 0, sem0)

        def body(i, carry):
            t = 2 * i
            fire(t + 1, 1, sem1)
            drain(t, 0, sem0)
            fire(t + 2, 0, sem0)
            drain(t + 1, 1, sem1)
            return carry

        lax.fori_loop(0, T // 2 - 1, body, 0)
        t = T - 2
        fire(t + 1, 1, sem1)
        drain(t, 0, sem0)
        drain(t + 1, 1, sem1)

    return gather_k


def _lstm_body(x_ref, wih_ref, whh_ref, b_ref, wfc_ref, bfc_ref, out_ref,
               h_ref, c_ref, *, H, T, D):
    t = pl.program_id(1)

    @pl.when(t == 0)
    def _():
        h_ref[...] = jnp.zeros_like(h_ref)
        c_ref[...] = jnp.zeros_like(c_ref)

    x = x_ref[0][:, 0:D]
    h = h_ref[...]
    g = (
        jnp.dot(x, wih_ref[...], preferred_element_type=jnp.float32)
        + jnp.dot(h, whh_ref[...], preferred_element_type=jnp.float32)
        + b_ref[...]
    )
    i_g = jax.nn.sigmoid(g[:, 0 * H:1 * H])
    f_g = jax.nn.sigmoid(g[:, 1 * H:2 * H])
    g_g = jnp.tanh(g[:, 2 * H:3 * H])
    o_g = jax.nn.sigmoid(g[:, 3 * H:4 * H])
    c_new = f_g * c_ref[...] + i_g * g_g
    h_new = o_g * jnp.tanh(c_new)
    c_ref[...] = c_new
    h_ref[...] = h_new

    @pl.when(t == T - 1)
    def _():
        logits = (
            jnp.dot(h_new, wfc_ref[...], preferred_element_type=jnp.float32)
            + bfc_ref[...]
        )
        m = jnp.max(logits, axis=-1, keepdims=True)
        s = logits - m
        lse = jnp.log(jnp.sum(jnp.exp(s), axis=-1, keepdims=True))
        out_ref[...] = s - lse


def _lstm_call(x, wih, whh, b1, wfc, bfc1, *, bt=2048):
    T, B, _ = x.shape
    D = wih.shape[0]
    H = whh.shape[0]
    A = wfc.shape[1]
    grid = (B // bt, T)
    return pl.pallas_call(
        functools.partial(_lstm_body, H=H, T=T, D=D),
        grid=grid,
        in_specs=[
            pl.BlockSpec((1, bt, 128), lambda b, t: (t, b, 0)),
            pl.BlockSpec((D, 4 * H), lambda b, t: (0, 0)),
            pl.BlockSpec((H, 4 * H), lambda b, t: (0, 0)),
            pl.BlockSpec((1, 4 * H), lambda b, t: (0, 0)),
            pl.BlockSpec((H, A), lambda b, t: (0, 0)),
            pl.BlockSpec((1, A), lambda b, t: (0, 0)),
        ],
        out_specs=pl.BlockSpec((bt, A), lambda b, t: (b, 0)),
        out_shape=jax.ShapeDtypeStruct((B, A), jnp.float32),
        scratch_shapes=[
            pltpu.VMEM((bt, H), jnp.float32),
            pltpu.VMEM((bt, H), jnp.float32),
        ],
        compiler_params=pltpu.CompilerParams(
            dimension_semantics=("parallel", "arbitrary"),
        ),
    )(x, wih, whh, b1, wfc, bfc1)


def kernel(dct_in, emb, W_ih, W_hh, b_ih, b_hh, W_fc, b_fc):
    B, T = dct_in.shape
    V, D = emb.shape
    H = W_hh.shape[1]
    A = W_fc.shape[0]

    embp = _expand_call(emb.T)
    idxt = dct_in.T.astype(jnp.int32)
    x3 = _make_sc_gather(V, B, T)(embp, idxt)

    b1 = (b_ih + b_hh).reshape(1, 4 * H)
    bfc1 = b_fc.reshape(1, A)
    return _lstm_call(x3, W_ih.T, W_hh.T, b1, W_fc.T, bfc1)


# speedup vs baseline: 3.2909x; 1.0271x over previous
import functools

import jax
import jax.numpy as jnp
from jax import lax
from jax.experimental import pallas as pl
from jax.experimental.pallas import tpu as pltpu
from jax.experimental.pallas import tpu_sc as plsc


def _expand_body(embt_ref, out_ref, *, D, C):
    x = embt_ref[...]
    xt = jnp.transpose(x, (1, 0))
    out_ref[...] = jnp.concatenate(
        [xt, jnp.zeros((C, 128 - D), jnp.float32)], axis=1
    )


def _expand_call(embt, *, C=16384):
    D, V = embt.shape
    return pl.pallas_call(
        functools.partial(_expand_body, D=D, C=C),
        grid=(pl.cdiv(V, C),),
        in_specs=[pl.BlockSpec((D, C), lambda i: (0, i))],
        out_specs=pl.BlockSpec((C, 128), lambda i: (i, 0)),
        out_shape=jax.ShapeDtypeStruct((V, 128), jnp.float32),
        compiler_params=pltpu.CompilerParams(
            dimension_semantics=("arbitrary",),
        ),
    )(embt)


@functools.lru_cache(maxsize=None)
def _make_sc_gather(V, B, T):
    info = plsc.get_sparse_core_info()
    NC, NS = info.num_cores, info.num_subcores
    NW = NC * NS
    BB = B // NW
    assert BB == 128
    mesh = plsc.VectorSubcoreMesh(core_axis_name="c", subcore_axis_name="s")

    @functools.partial(
        pl.kernel,
        mesh=mesh,
        out_type=jax.ShapeDtypeStruct((T, B, 128), jnp.float32),
        scratch_types=[
            pltpu.VMEM((T, BB), jnp.int32),
            pltpu.VMEM((2, BB, 128), jnp.float32),
            pltpu.SemaphoreType.DMA,
            pltpu.SemaphoreType.DMA,
        ],
        compiler_params=pltpu.CompilerParams(use_tc_tiling_on_sc=False),
    )
    def gather_k(embp_hbm, idxt_hbm, out_hbm, idx_v, rows_v, sem0, sem1):
        wid = lax.axis_index("s") * NC + lax.axis_index("c")
        b0 = wid * BB
        pltpu.sync_copy(idxt_hbm.at[:, pl.ds(b0, BB)], idx_v)

        def fire(t, buf, sem):
            pltpu.async_copy(embp_hbm.at[idx_v.at[t]], rows_v.at[buf], sem)

        def drain(t, buf, sem):
            pltpu.make_async_copy(
                embp_hbm.at[idx_v.at[t]], rows_v.at[buf], sem
            ).wait()
            pltpu.sync_copy(rows_v.at[buf], out_hbm.at[t, pl.ds(b0, BB)])

        fire(0, 0, sem0)

        def body(i, carry):
            t = 2 * i
            fire(t + 1, 1, sem1)
            drain(t, 0, sem0)
            fire(t + 2, 0, sem0)
            drain(t + 1, 1, sem1)
            return carry

        lax.fori_loop(0, T // 2 - 1, body, 0)
        t = T - 2
        fire(t + 1, 1, sem1)
        drain(t, 0, sem0)
        drain(t + 1, 1, sem1)

    return gather_k


def _lstm_body(x_ref, wih_ref, whh_ref, b_ref, wfc_ref, bfc_ref, out_ref,
               h_ref, c_ref, *, H, T, D):
    t = pl.program_id(1)

    @pl.when(t == 0)
    def _():
        h_ref[...] = jnp.zeros_like(h_ref)
        c_ref[...] = jnp.zeros_like(c_ref)

    x = x_ref[0][:, 0:D]
    h = h_ref[...]
    g = (
        jnp.dot(x, wih_ref[...], preferred_element_type=jnp.float32)
        + jnp.dot(h, whh_ref[...], preferred_element_type=jnp.float32)
        + b_ref[...]
    )
    i_g = jax.nn.sigmoid(g[:, 0 * H:1 * H])
    f_g = jax.nn.sigmoid(g[:, 1 * H:2 * H])
    g_g = jnp.tanh(g[:, 2 * H:3 * H])
    o_g = jax.nn.sigmoid(g[:, 3 * H:4 * H])
    c_new = f_g * c_ref[...] + i_g * g_g
    h_new = o_g * jnp.tanh(c_new)
    c_ref[...] = c_new
    h_ref[...] = h_new

    @pl.when(t == T - 1)
    def _():
        logits = (
            jnp.dot(h_new, wfc_ref[...], preferred_element_type=jnp.float32)
            + bfc_ref[...]
        )
        m = jnp.max(logits, axis=-1, keepdims=True)
        s = logits - m
        lse = jnp.log(jnp.sum(jnp.exp(s), axis=-1, keepdims=True))
        out_ref[...] = s - lse


def _lstm_call(x, wih, whh, b1, wfc, bfc1, *, bt=4096):
    T, B, _ = x.shape
    D = wih.shape[0]
    H = whh.shape[0]
    A = wfc.shape[1]
    grid = (B // bt, T)
    return pl.pallas_call(
        functools.partial(_lstm_body, H=H, T=T, D=D),
        grid=grid,
        in_specs=[
            pl.BlockSpec((1, bt, 128), lambda b, t: (t, b, 0)),
            pl.BlockSpec((D, 4 * H), lambda b, t: (0, 0)),
            pl.BlockSpec((H, 4 * H), lambda b, t: (0, 0)),
            pl.BlockSpec((1, 4 * H), lambda b, t: (0, 0)),
            pl.BlockSpec((H, A), lambda b, t: (0, 0)),
            pl.BlockSpec((1, A), lambda b, t: (0, 0)),
        ],
        out_specs=pl.BlockSpec((bt, A), lambda b, t: (b, 0)),
        out_shape=jax.ShapeDtypeStruct((B, A), jnp.float32),
        scratch_shapes=[
            pltpu.VMEM((bt, H), jnp.float32),
            pltpu.VMEM((bt, H), jnp.float32),
        ],
        compiler_params=pltpu.CompilerParams(
            dimension_semantics=("parallel", "arbitrary"),
        ),
    )(x, wih, whh, b1, wfc, bfc1)


def kernel(dct_in, emb, W_ih, W_hh, b_ih, b_hh, W_fc, b_fc):
    B, T = dct_in.shape
    V, D = emb.shape
    H = W_hh.shape[1]
    A = W_fc.shape[0]

    embp = _expand_call(emb.T)
    idxt = dct_in.T.astype(jnp.int32)
    x3 = _make_sc_gather(V, B, T)(embp, idxt)

    b1 = (b_ih + b_hh).reshape(1, 4 * H)
    bfc1 = b_fc.reshape(1, A)
    return _lstm_call(x3, W_ih.T, W_hh.T, b1, W_fc.T, bfc1)
